# baseline (speedup 1.0000x reference)
import functools

import jax
import jax.numpy as jnp
from jax import lax
from jax.experimental import pallas as pl
from jax.experimental.pallas import tpu as pltpu
from jax.experimental.pallas import tpu_sc as plsc

B = 16384
DENSE_DIM = 64
DENSE_EMB = 16
SPARSE_VOCAB = 1000000
SPARSE_EMB = 64
OUT = 128

NC = 2
NS = 16
NW = NC * NS
B_PER_W = B // NW
IDX_CHUNK = 128
N_CHUNKS = B_PER_W // IDX_CHUNK
GPAD = 128

LB = 32768
PACKED = SPARSE_EMB // 2
VPL = 128 // PACKED
LB4 = LB // VPL
NBLK = (SPARSE_VOCAB + LB - 1) // LB
NLINES = NBLK * LB4


def _transpose_body(src_ref, dst_ref):
    s = src_ref[...]
    lo = lax.bitcast_convert_type(
        s[:PACKED, :].astype(jnp.bfloat16), jnp.uint16).astype(jnp.uint32)
    hi = lax.bitcast_convert_type(
        s[PACKED:, :].astype(jnp.bfloat16), jnp.uint16).astype(jnp.uint32)
    packed = lax.bitcast_convert_type((hi << 16) | lo, jnp.int32)
    pt = packed.T
    dst_ref[...] = jnp.concatenate(
        [pt[w * LB4:(w + 1) * LB4, :] for w in range(VPL)], axis=1)


def _transpose_table(table_t):
    return pl.pallas_call(
        _transpose_body,
        grid=(NBLK,),
        in_specs=[pl.BlockSpec((SPARSE_EMB, LB), lambda i: (0, i))],
        out_specs=pl.BlockSpec((LB4, 128), lambda i: (i, 0)),
        out_shape=jax.ShapeDtypeStruct((NLINES, 128), jnp.int32),
        compiler_params=pltpu.CompilerParams(
            dimension_semantics=("parallel",),
            vmem_limit_bytes=100 * 1024 * 1024),
    )(table_t)


@functools.cache
def _make_sc_gather():
    mesh = plsc.VectorSubcoreMesh(core_axis_name="c", subcore_axis_name="s")

    @functools.partial(
        pl.kernel,
        mesh=mesh,
        out_type=jax.ShapeDtypeStruct((B, GPAD), jnp.int32),
        scratch_types=[
            pltpu.VMEM((N_CHUNKS, IDX_CHUNK), jnp.int32),
            pltpu.VMEM((B_PER_W, GPAD), jnp.int32),
            pltpu.SemaphoreType.DMA,
        ],
        compiler_params=pltpu.CompilerParams(use_tc_tiling_on_sc=True,
                                             needs_layout_passes=False),
    )
    def _sc_gather(table_hbm, idx_hbm, out_hbm, idx_v, rows_v, sem):
        wid = lax.axis_index("s") * NC + lax.axis_index("c")
        pltpu.sync_copy(idx_hbm.at[pl.ds(wid * N_CHUNKS, N_CHUNKS)], idx_v)
        for j in range(N_CHUNKS):

            def group(g, _, j=j):
                v = idx_v[j, pl.ds(g * 16, 16)]
                vq = ((v >> 15) << 13) | (v & (LB4 - 1))
                base = j * IDX_CHUNK + g * 16
                copies = [
                    pltpu.async_copy(
                        table_hbm.at[vq[k]],
                        rows_v.at[base + k],
                        sem,
                    )
                    for k in range(16)
                ]
                for cp in copies:
                    cp.wait()
                return _

            lax.fori_loop(0, IDX_CHUNK // 16, group, 0)
        pltpu.sync_copy(rows_v, out_hbm.at[pl.ds(wid * B_PER_W, B_PER_W)])

    return _sc_gather


_RB = 2048


def _tc_body(g_ref, sidx_ref, didx_ref, dtab_ref, w_ref, b_ref, o_ref):
    wsel = (sidx_ref[...] >> 13) & 3
    gfull = g_ref[...]
    gi = lax.bitcast_convert_type(
        jnp.where(wsel == 0, gfull[:, 0:32],
                  jnp.where(wsel == 1, gfull[:, 32:64],
                            jnp.where(wsel == 2, gfull[:, 64:96],
                                      gfull[:, 96:128]))), jnp.uint32)
    lo = lax.bitcast_convert_type(
        (gi & 0xFFFF).astype(jnp.uint16), jnp.bfloat16).astype(jnp.float32)
    hi = lax.bitcast_convert_type(
        (gi >> 16).astype(jnp.uint16), jnp.bfloat16).astype(jnp.float32)
    g = jnp.concatenate([lo, hi], axis=1)
    didx = didx_ref[...]
    onehot = (lax.broadcasted_iota(jnp.int32, (_RB, DENSE_DIM), 1)
              == didx).astype(jnp.float32)
    w = w_ref[...]
    w_sparse = w[:SPARSE_EMB, :]
    w_dense = w[SPARSE_EMB:, :]
    fused = jnp.dot(dtab_ref[...], w_dense,
                    preferred_element_type=jnp.float32)
    acc = (jnp.dot(g, w_sparse, preferred_element_type=jnp.float32)
           + jnp.dot(onehot, fused, preferred_element_type=jnp.float32)
           + b_ref[...])
    o_ref[...] = jnp.maximum(acc, 0.0)


def kernel(dense_input, sparse_input, dense_table, sparse_table, W, b):
    sparse_idx = sparse_input.reshape(B // IDX_CHUNK, IDX_CHUNK)
    table_rm = _transpose_table(sparse_table.T)
    gathered = _make_sc_gather()(table_rm, sparse_idx)

    grid = B // _RB
    out = pl.pallas_call(
        _tc_body,
        grid=(grid,),
        in_specs=[
            pl.BlockSpec((_RB, GPAD), lambda i: (i, 0)),
            pl.BlockSpec((_RB, 1), lambda i: (i, 0)),
            pl.BlockSpec((_RB, 1), lambda i: (i, 0)),
            pl.BlockSpec((DENSE_DIM, DENSE_EMB), lambda i: (0, 0)),
            pl.BlockSpec((SPARSE_EMB + DENSE_EMB, OUT), lambda i: (0, 0)),
            pl.BlockSpec((1, OUT), lambda i: (0, 0)),
        ],
        out_specs=pl.BlockSpec((_RB, OUT), lambda i: (i, 0)),
        out_shape=jax.ShapeDtypeStruct((B, OUT), jnp.float32),
        compiler_params=pltpu.CompilerParams(
            dimension_semantics=("parallel",)),
    )(gathered, sparse_input, dense_input, dense_table, W,
      b.reshape(1, OUT))
    return out


# speedup vs baseline: 1.0542x; 1.0542x over previous
import functools

import jax
import jax.numpy as jnp
from jax import lax
from jax.experimental import pallas as pl
from jax.experimental.pallas import tpu as pltpu
from jax.experimental.pallas import tpu_sc as plsc

B = 16384
DENSE_DIM = 64
DENSE_EMB = 16
SPARSE_VOCAB = 1000000
SPARSE_EMB = 64
OUT = 128

NC = 2
NS = 16
NW = NC * NS
B_PER_W = B // NW
IDX_CHUNK = 128
N_CHUNKS = B_PER_W // IDX_CHUNK
GPAD = 128

LB = 32768
PACKED = SPARSE_EMB // 2


def _transpose_body(src_ref, dst_ref):
    s = src_ref[...]
    lo = lax.bitcast_convert_type(
        s[:PACKED, :].astype(jnp.bfloat16), jnp.uint16).astype(jnp.uint32)
    hi = lax.bitcast_convert_type(
        s[PACKED:, :].astype(jnp.bfloat16), jnp.uint16).astype(jnp.uint32)
    packed = lax.bitcast_convert_type((hi << 16) | lo, jnp.int32)
    dst_ref[...] = packed.T


def _transpose_table(table_t):
    grid = (SPARSE_VOCAB + LB - 1) // LB
    return pl.pallas_call(
        _transpose_body,
        grid=(grid,),
        in_specs=[pl.BlockSpec((SPARSE_EMB, LB), lambda i: (0, i))],
        out_specs=pl.BlockSpec((LB, PACKED), lambda i: (i, 0)),
        out_shape=jax.ShapeDtypeStruct((SPARSE_VOCAB, PACKED), jnp.int32),
        compiler_params=pltpu.CompilerParams(
            dimension_semantics=("parallel",),
            vmem_limit_bytes=100 * 1024 * 1024),
    )(table_t)


@functools.cache
def _make_sc_gather():
    mesh = plsc.VectorSubcoreMesh(core_axis_name="c", subcore_axis_name="s")

    @functools.partial(
        pl.kernel,
        mesh=mesh,
        out_type=jax.ShapeDtypeStruct((B, GPAD), jnp.int32),
        scratch_types=[
            pltpu.VMEM((N_CHUNKS, IDX_CHUNK), jnp.int32),
            pltpu.VMEM((B_PER_W, GPAD), jnp.int32),
            pltpu.SemaphoreType.DMA,
        ],
        compiler_params=pltpu.CompilerParams(use_tc_tiling_on_sc=True,
                                             needs_layout_passes=False),
    )
    def _sc_gather(table_hbm, idx_hbm, out_hbm, idx_v, rows_v, sem):
        wid = lax.axis_index("s") * NC + lax.axis_index("c")
        pltpu.sync_copy(idx_hbm.at[pl.ds(wid * N_CHUNKS, N_CHUNKS)], idx_v)
        for j in range(N_CHUNKS):

            def group(g, _, j=j):
                v = idx_v[j, pl.ds(g * 16, 16)]
                base = j * IDX_CHUNK + g * 16
                copies = [
                    pltpu.async_copy(
                        table_hbm.at[v[k]],
                        rows_v.at[base + k, pl.ds(0, PACKED)],
                        sem,
                    )
                    for k in range(16)
                ]
                for cp in copies:
                    cp.wait()
                return _

            lax.fori_loop(0, IDX_CHUNK // 16, group, 0)
        pltpu.sync_copy(rows_v, out_hbm.at[pl.ds(wid * B_PER_W, B_PER_W)])

    return _sc_gather


_RB = 2048


def _tc_body(g_ref, didx_ref, dtab_ref, w_ref, b_ref, o_ref):
    gi = lax.bitcast_convert_type(g_ref[:, :PACKED], jnp.uint32)
    lo = lax.bitcast_convert_type(
        (gi & 0xFFFF).astype(jnp.uint16), jnp.bfloat16).astype(jnp.float32)
    hi = lax.bitcast_convert_type(
        (gi >> 16).astype(jnp.uint16), jnp.bfloat16).astype(jnp.float32)
    g = jnp.concatenate([lo, hi], axis=1)
    didx = didx_ref[...]
    onehot = (lax.broadcasted_iota(jnp.int32, (_RB, DENSE_DIM), 1)
              == didx).astype(jnp.float32)
    w = w_ref[...]
    w_sparse = w[:SPARSE_EMB, :]
    w_dense = w[SPARSE_EMB:, :]
    fused = jnp.dot(dtab_ref[...], w_dense,
                    preferred_element_type=jnp.float32)
    acc = (jnp.dot(g, w_sparse, preferred_element_type=jnp.float32)
           + jnp.dot(onehot, fused, preferred_element_type=jnp.float32)
           + b_ref[...])
    o_ref[...] = jnp.maximum(acc, 0.0)


def kernel(dense_input, sparse_input, dense_table, sparse_table, W, b):
    sparse_idx = sparse_input.reshape(B // IDX_CHUNK, IDX_CHUNK)
    table_rm = _transpose_table(sparse_table.T)
    gathered = _make_sc_gather()(table_rm, sparse_idx)

    grid = B // _RB
    out = pl.pallas_call(
        _tc_body,
        grid=(grid,),
        in_specs=[
            pl.BlockSpec((_RB, GPAD), lambda i: (i, 0)),
            pl.BlockSpec((_RB, 1), lambda i: (i, 0)),
            pl.BlockSpec((DENSE_DIM, DENSE_EMB), lambda i: (0, 0)),
            pl.BlockSpec((SPARSE_EMB + DENSE_EMB, OUT), lambda i: (0, 0)),
            pl.BlockSpec((1, OUT), lambda i: (0, 0)),
        ],
        out_specs=pl.BlockSpec((_RB, OUT), lambda i: (i, 0)),
        out_shape=jax.ShapeDtypeStruct((B, OUT), jnp.float32),
        compiler_params=pltpu.CompilerParams(
            dimension_semantics=("parallel",)),
    )(gathered, dense_input, dense_table, W, b.reshape(1, OUT))
    return out


# speedup vs baseline: 1.0657x; 1.0108x over previous
import functools

import jax
import jax.numpy as jnp
from jax import lax
from jax.experimental import pallas as pl
from jax.experimental.pallas import tpu as pltpu
from jax.experimental.pallas import tpu_sc as plsc

B = 16384
DENSE_DIM = 64
DENSE_EMB = 16
SPARSE_VOCAB = 1000000
SPARSE_EMB = 64
OUT = 128

NC = 2
NS = 16
NW = NC * NS
B_PER_W = B // NW
IDX_CHUNK = 128
N_CHUNKS = B_PER_W // IDX_CHUNK
GPAD = 128

LB = 32768
PACKED = SPARSE_EMB // 2


def _transpose_body(src_ref, dst_ref):
    s = src_ref[...]
    lo = lax.bitcast_convert_type(
        s[:PACKED, :].astype(jnp.bfloat16), jnp.uint16).astype(jnp.uint32)
    hi = lax.bitcast_convert_type(
        s[PACKED:, :].astype(jnp.bfloat16), jnp.uint16).astype(jnp.uint32)
    packed = lax.bitcast_convert_type((hi << 16) | lo, jnp.int32)
    dst_ref[...] = packed.T


def _transpose_table(table_t):
    grid = (SPARSE_VOCAB + LB - 1) // LB
    return pl.pallas_call(
        _transpose_body,
        grid=(grid,),
        in_specs=[pl.BlockSpec((SPARSE_EMB, LB), lambda i: (0, i))],
        out_specs=pl.BlockSpec((LB, PACKED), lambda i: (i, 0)),
        out_shape=jax.ShapeDtypeStruct((SPARSE_VOCAB, PACKED), jnp.int32),
        compiler_params=pltpu.CompilerParams(
            dimension_semantics=("parallel",),
            vmem_limit_bytes=100 * 1024 * 1024),
    )(table_t)


@functools.cache
def _make_sc_gather():
    mesh = plsc.VectorSubcoreMesh(core_axis_name="c", subcore_axis_name="s")

    @functools.partial(
        pl.kernel,
        mesh=mesh,
        out_type=jax.ShapeDtypeStruct((B, GPAD), jnp.int32),
        scratch_types=[
            pltpu.VMEM((N_CHUNKS, IDX_CHUNK), jnp.int32),
            pltpu.VMEM((B_PER_W, GPAD), jnp.int32),
            pltpu.SemaphoreType.DMA,
        ],
        compiler_params=pltpu.CompilerParams(use_tc_tiling_on_sc=True,
                                             needs_layout_passes=False),
    )
    def _sc_gather(table_hbm, idx_hbm, out_hbm, idx_v, rows_v, sem):
        wid = lax.axis_index("s") * NC + lax.axis_index("c")
        pltpu.sync_copy(idx_hbm.at[pl.ds(wid * N_CHUNKS, N_CHUNKS)], idx_v)
        prev = None
        for j in range(N_CHUNKS):
            for g in range(IDX_CHUNK // 16):
                v = idx_v[j, pl.ds(g * 16, 16)]
                base = j * IDX_CHUNK + g * 16
                copies = [
                    pltpu.async_copy(
                        table_hbm.at[v[k]],
                        rows_v.at[base + k, pl.ds(0, PACKED)],
                        sem,
                    )
                    for k in range(16)
                ]
                if prev is not None:
                    for cp in prev:
                        cp.wait()
                prev = copies
        for cp in prev:
            cp.wait()
        pltpu.sync_copy(rows_v, out_hbm.at[pl.ds(wid * B_PER_W, B_PER_W)])

    return _sc_gather


_RB = 2048


def _tc_body(g_ref, didx_ref, dtab_ref, w_ref, b_ref, o_ref):
    gi = lax.bitcast_convert_type(g_ref[:, :PACKED], jnp.uint32)
    lo = lax.bitcast_convert_type(
        (gi & 0xFFFF).astype(jnp.uint16), jnp.bfloat16).astype(jnp.float32)
    hi = lax.bitcast_convert_type(
        (gi >> 16).astype(jnp.uint16), jnp.bfloat16).astype(jnp.float32)
    g = jnp.concatenate([lo, hi], axis=1)
    didx = didx_ref[...]
    onehot = (lax.broadcasted_iota(jnp.int32, (_RB, DENSE_DIM), 1)
              == didx).astype(jnp.float32)
    w = w_ref[...]
    w_sparse = w[:SPARSE_EMB, :]
    w_dense = w[SPARSE_EMB:, :]
    fused = jnp.dot(dtab_ref[...], w_dense,
                    preferred_element_type=jnp.float32)
    acc = (jnp.dot(g, w_sparse, preferred_element_type=jnp.float32)
           + jnp.dot(onehot, fused, preferred_element_type=jnp.float32)
           + b_ref[...])
    o_ref[...] = jnp.maximum(acc, 0.0)


def kernel(dense_input, sparse_input, dense_table, sparse_table, W, b):
    sparse_idx = sparse_input.reshape(B // IDX_CHUNK, IDX_CHUNK)
    table_rm = _transpose_table(sparse_table.T)
    gathered = _make_sc_gather()(table_rm, sparse_idx)

    grid = B // _RB
    out = pl.pallas_call(
        _tc_body,
        grid=(grid,),
        in_specs=[
            pl.BlockSpec((_RB, GPAD), lambda i: (i, 0)),
            pl.BlockSpec((_RB, 1), lambda i: (i, 0)),
            pl.BlockSpec((DENSE_DIM, DENSE_EMB), lambda i: (0, 0)),
            pl.BlockSpec((SPARSE_EMB + DENSE_EMB, OUT), lambda i: (0, 0)),
            pl.BlockSpec((1, OUT), lambda i: (0, 0)),
        ],
        out_specs=pl.BlockSpec((_RB, OUT), lambda i: (i, 0)),
        out_shape=jax.ShapeDtypeStruct((B, OUT), jnp.float32),
        compiler_params=pltpu.CompilerParams(
            dimension_semantics=("parallel",)),
    )(gathered, dense_input, dense_table, W, b.reshape(1, OUT))
    return out
